# baseline (speedup 1.0000x reference)
import functools

import jax
import jax.numpy as jnp
from jax import lax
from jax.experimental import pallas as pl
from jax.experimental.pallas import tpu as pltpu
from jax.experimental.pallas import tpu_sc as plsc

NODE_DIM = 128
EDGE_DIM = 16
HIDDEN_DIM = 128
N = 10000
E = 320000

NC = 2
NS = 16
NW = NC * NS
PER_W = E // NW
CHUNK = 128
K = 79
TAIL = PER_W - (K - 1) * CHUNK
PADW = K * CHUNK
PADK = 80
NPAIR = 40
N_PAD = 10240
ROWS_PER_SUB = N_PAD // NS

_sc_mesh = plsc.VectorSubcoreMesh(core_axis_name="c", subcore_axis_name="s")
_sc_params = pltpu.CompilerParams(use_tc_tiling_on_sc=False)


def _proj_body(nf_ref, w_ref, psrc_ref, pdst_ref):
    nf = nf_ref[...]
    psrc_ref[...] = jnp.dot(nf, w_ref[0:NODE_DIM, :],
                            preferred_element_type=jnp.float32)
    pdst_ref[...] = jnp.dot(nf, w_ref[NODE_DIM:2 * NODE_DIM, :],
                            preferred_element_type=jnp.float32)


def _project_nodes(nf, eW1ab):
    return pl.pallas_call(
        _proj_body,
        out_shape=(
            jax.ShapeDtypeStruct((N, HIDDEN_DIM), jnp.float32),
            jax.ShapeDtypeStruct((N, HIDDEN_DIM), jnp.float32),
        ),
    )(nf, eW1ab)


@functools.partial(
    pl.kernel,
    out_type=(
        jax.ShapeDtypeStruct((E, HIDDEN_DIM), jnp.float32),
        jax.ShapeDtypeStruct((E, HIDDEN_DIM), jnp.float32),
    ),
    mesh=_sc_mesh,
    scratch_types=[
        pltpu.VMEM((PADK, CHUNK), jnp.int32),
        pltpu.VMEM((PADK, CHUNK), jnp.int32),
        pltpu.VMEM((CHUNK, HIDDEN_DIM), jnp.float32),
        pltpu.VMEM((CHUNK, HIDDEN_DIM), jnp.float32),
        pltpu.VMEM((CHUNK, HIDDEN_DIM), jnp.float32),
        pltpu.VMEM((CHUNK, HIDDEN_DIM), jnp.float32),
        pltpu.SemaphoreType.DMA,
        pltpu.SemaphoreType.DMA,
        pltpu.SemaphoreType.DMA,
        pltpu.SemaphoreType.DMA,
    ],
)
def _sc_gather(psrc_hbm, pdst_hbm, sidx_hbm, didx_hbm, gsrc_hbm, gdst_hbm,
               sidx_v, didx_v, a0, a1, b0, b1, sa0, sa1, sb0, sb1):
    wid = lax.axis_index("s") * NC + lax.axis_index("c")
    base = wid * PER_W
    pltpu.sync_copy(sidx_hbm.at[pl.ds(wid * PADK, PADK)], sidx_v)
    pltpu.sync_copy(didx_hbm.at[pl.ds(wid * PADK, PADK)], didx_v)

    def start(j, abuf, bbuf, sa, sb):
        pltpu.async_copy(psrc_hbm.at[sidx_v.at[j]], abuf, sa)
        pltpu.async_copy(pdst_hbm.at[didx_v.at[j]], bbuf, sb)

    def drain(j, abuf, bbuf, sa, sb):
        pltpu.make_async_copy(psrc_hbm.at[sidx_v.at[j]], abuf, sa).wait()

        @pl.when(j == K - 1)
        def _():
            pltpu.sync_copy(abuf.at[pl.ds(0, TAIL)],
                            gsrc_hbm.at[pl.ds(base + j * CHUNK, TAIL)])

        @pl.when(j != K - 1)
        def _():
            pltpu.sync_copy(abuf, gsrc_hbm.at[pl.ds(base + j * CHUNK, CHUNK)])

        pltpu.make_async_copy(pdst_hbm.at[didx_v.at[j]], bbuf, sb).wait()

        @pl.when(j == K - 1)
        def _():
            pltpu.sync_copy(bbuf.at[pl.ds(0, TAIL)],
                            gdst_hbm.at[pl.ds(base + j * CHUNK, TAIL)])

        @pl.when(j != K - 1)
        def _():
            pltpu.sync_copy(bbuf, gdst_hbm.at[pl.ds(base + j * CHUNK, CHUNK)])

    start(0, a0, b0, sa0, sb0)

    def pair(t, _):
        j0 = 2 * t

        @pl.when(t < NPAIR - 1)
        def _():
            start(j0 + 1, a1, b1, sa1, sb1)

        drain(j0, a0, b0, sa0, sb0)

        @pl.when(t < NPAIR - 1)
        def _():
            start(j0 + 2, a0, b0, sa0, sb0)
            drain(j0 + 1, a1, b1, sa1, sb1)

        return 0

    lax.fori_loop(0, NPAIR, pair, 0)


def _edge_body(gsrc_ref, gdst_ref, ef_ref, w1c_ref, b1_ref, w2_ref, b2_ref,
               g_ref, bt_ref, eo_ref):
    ef = ef_ref[...]
    pre = jnp.dot(ef, w1c_ref[...], preferred_element_type=jnp.float32)
    z = gsrc_ref[...] + gdst_ref[...] + pre + b1_ref[...]
    h1 = z * jax.nn.sigmoid(z)
    h = jnp.dot(h1, w2_ref[...], preferred_element_type=jnp.float32) + b2_ref[...]
    x = ef + h
    ones_m = jnp.full((EDGE_DIM, EDGE_DIM), 1.0 / EDGE_DIM, jnp.float32)
    m = jnp.dot(x, ones_m, preferred_element_type=jnp.float32)
    x2m = jnp.dot(x * x, ones_m, preferred_element_type=jnp.float32)
    v = x2m - m * m
    eo_ref[...] = (x - m) * lax.rsqrt(v + 1e-5) * g_ref[...] + bt_ref[...]


_EB = 4000


def _edge_mlp(gsrc, gdst, ef, eW1c, eb1, eW2, eb2, eg, ebt):
    nb = E // _EB
    return pl.pallas_call(
        _edge_body,
        grid=(nb,),
        in_specs=[
            pl.BlockSpec((_EB, HIDDEN_DIM), lambda i: (i, 0)),
            pl.BlockSpec((_EB, HIDDEN_DIM), lambda i: (i, 0)),
            pl.BlockSpec((_EB, EDGE_DIM), lambda i: (i, 0)),
            pl.BlockSpec((EDGE_DIM, HIDDEN_DIM), lambda i: (0, 0)),
            pl.BlockSpec((1, HIDDEN_DIM), lambda i: (0, 0)),
            pl.BlockSpec((HIDDEN_DIM, EDGE_DIM), lambda i: (0, 0)),
            pl.BlockSpec((1, EDGE_DIM), lambda i: (0, 0)),
            pl.BlockSpec((1, EDGE_DIM), lambda i: (0, 0)),
            pl.BlockSpec((1, EDGE_DIM), lambda i: (0, 0)),
        ],
        out_specs=pl.BlockSpec((_EB, EDGE_DIM), lambda i: (i, 0)),
        out_shape=jax.ShapeDtypeStruct((E, EDGE_DIM), jnp.float32),
    )(gsrc, gdst, ef, eW1c, eb1, eW2, eb2, eg, ebt)


@functools.partial(
    pl.kernel,
    out_type=jax.ShapeDtypeStruct((NC, N_PAD, EDGE_DIM), jnp.float32),
    mesh=_sc_mesh,
    scratch_types=[
        pltpu.VMEM((PADK, CHUNK), jnp.int32),
        pltpu.VMEM((CHUNK, EDGE_DIM), jnp.float32),
        pltpu.VMEM((CHUNK, EDGE_DIM), jnp.float32),
        pltpu.VMEM_SHARED((N_PAD, EDGE_DIM), jnp.float32),
        pltpu.SemaphoreType.DMA,
        pltpu.SemaphoreType.DMA,
    ],
    compiler_params=_sc_params,
)
def _sc_scatter(eo_hbm, didx_hbm, zeros_hbm, agg_hbm, didx_v, r0, r1, table,
                s0, s1):
    c = lax.axis_index("c")
    s = lax.axis_index("s")
    wid = s * NC + c
    base = wid * PER_W
    pltpu.sync_copy(zeros_hbm.at[pl.ds(s * ROWS_PER_SUB, ROWS_PER_SUB)],
                    table.at[pl.ds(s * ROWS_PER_SUB, ROWS_PER_SUB)])
    pltpu.sync_copy(didx_hbm.at[pl.ds(wid * PADK, PADK)], didx_v)
    plsc.subcore_barrier()

    def start(j, rbuf, sem):
        @pl.when(j == K - 1)
        def _():
            pltpu.async_copy(eo_hbm.at[pl.ds(base + j * CHUNK, TAIL)],
                             rbuf.at[pl.ds(0, TAIL)], sem)

        @pl.when(j != K - 1)
        def _():
            pltpu.async_copy(eo_hbm.at[pl.ds(base + j * CHUNK, CHUNK)],
                             rbuf, sem)

    def drain(j, rbuf, sem):
        @pl.when(j == K - 1)
        def _():
            pltpu.make_async_copy(eo_hbm.at[pl.ds(base + j * CHUNK, TAIL)],
                                  rbuf.at[pl.ds(0, TAIL)], sem).wait()

        @pl.when(j != K - 1)
        def _():
            pltpu.make_async_copy(eo_hbm.at[pl.ds(base + j * CHUNK, CHUNK)],
                                  rbuf, sem).wait()

        pltpu.sync_copy(rbuf, table.at[didx_v.at[j]], add=True)

    start(0, r0, s0)

    def pair(t, _):
        j0 = 2 * t

        @pl.when(t < NPAIR - 1)
        def _():
            start(j0 + 1, r1, s1)

        drain(j0, r0, s0)

        @pl.when(t < NPAIR - 1)
        def _():
            start(j0 + 2, r0, s0)
            drain(j0 + 1, r1, s1)

        return 0

    lax.fori_loop(0, NPAIR, pair, 0)
    plsc.subcore_barrier()
    pltpu.sync_copy(table.at[pl.ds(s * ROWS_PER_SUB, ROWS_PER_SUB)],
                    agg_hbm.at[c, pl.ds(s * ROWS_PER_SUB, ROWS_PER_SUB)])


def _node_body(nf_ref, a0_ref, a1_ref, w1a_ref, w1b_ref, b1_ref, w2_ref,
               b2_ref, g_ref, bt_ref, out_ref):
    nf = nf_ref[...]
    agg = a0_ref[...] + a1_ref[...]
    z = (jnp.dot(nf, w1a_ref[...], preferred_element_type=jnp.float32)
         + jnp.dot(agg, w1b_ref[...], preferred_element_type=jnp.float32)
         + b1_ref[...])
    h1 = z * jax.nn.sigmoid(z)
    h2 = jnp.dot(h1, w2_ref[...], preferred_element_type=jnp.float32) + b2_ref[...]
    x = nf + h2
    ones_m = jnp.full((NODE_DIM, NODE_DIM), 1.0 / NODE_DIM, jnp.float32)
    m = jnp.dot(x, ones_m, preferred_element_type=jnp.float32)
    x2m = jnp.dot(x * x, ones_m, preferred_element_type=jnp.float32)
    v = x2m - m * m
    out_ref[...] = (x - m) * lax.rsqrt(v + 1e-5) * g_ref[...] + bt_ref[...]


_NB = 2000


def _node_mlp(nf, a0, a1, nW1a, nW1b, nb1, nW2, nb2, ng, nbt):
    nb = N // _NB
    return pl.pallas_call(
        _node_body,
        grid=(nb,),
        in_specs=[
            pl.BlockSpec((_NB, NODE_DIM), lambda i: (i, 0)),
            pl.BlockSpec((_NB, EDGE_DIM), lambda i: (i, 0)),
            pl.BlockSpec((_NB, EDGE_DIM), lambda i: (i, 0)),
            pl.BlockSpec((NODE_DIM, HIDDEN_DIM), lambda i: (0, 0)),
            pl.BlockSpec((EDGE_DIM, HIDDEN_DIM), lambda i: (0, 0)),
            pl.BlockSpec((1, HIDDEN_DIM), lambda i: (0, 0)),
            pl.BlockSpec((HIDDEN_DIM, NODE_DIM), lambda i: (0, 0)),
            pl.BlockSpec((1, NODE_DIM), lambda i: (0, 0)),
            pl.BlockSpec((1, NODE_DIM), lambda i: (0, 0)),
            pl.BlockSpec((1, NODE_DIM), lambda i: (0, 0)),
        ],
        out_specs=pl.BlockSpec((_NB, NODE_DIM), lambda i: (i, 0)),
        out_shape=jax.ShapeDtypeStruct((N, NODE_DIM), jnp.float32),
    )(nf, a0, a1, nW1a, nW1b, nb1, nW2, nb2, ng, nbt)


def _worker_idx(idx, pad_value):
    arr = idx.reshape(NW, PER_W)
    arr = jnp.pad(arr, ((0, 0), (0, PADK * CHUNK - PER_W)),
                  constant_values=pad_value)
    return arr.reshape(NW * PADK, CHUNK)


@jax.jit
def kernel(node_feat, edge_feat, edge_index, eW1, eb1, eW2, eb2, eg, ebt,
           nW1, nb1, nW2, nb2, ng, nbt):
    nf = node_feat[0]
    ef = edge_feat[0]
    src = edge_index[0].astype(jnp.int32)
    dst = edge_index[1].astype(jnp.int32)

    sidx = _worker_idx(src, 0)
    didx_g = _worker_idx(dst, 0)
    didx_s = _worker_idx(dst, N)

    psrc, pdst = _project_nodes(nf, eW1[0:2 * NODE_DIM])
    gsrc, gdst = _sc_gather(psrc, pdst, sidx, didx_g)
    eo = _edge_mlp(gsrc, gdst, ef, eW1[2 * NODE_DIM:],
                   eb1.reshape(1, -1), eW2, eb2.reshape(1, -1),
                   eg.reshape(1, -1), ebt.reshape(1, -1))
    zeros = jnp.zeros((N_PAD, EDGE_DIM), jnp.float32)
    agg2 = _sc_scatter(eo, didx_s, zeros)
    node_out = _node_mlp(nf, agg2[0, :N], agg2[1, :N],
                         nW1[0:NODE_DIM], nW1[NODE_DIM:],
                         nb1.reshape(1, -1), nW2, nb2.reshape(1, -1),
                         ng.reshape(1, -1), nbt.reshape(1, -1))
    return (node_out[None], eo[None])


# speedup vs baseline: 1.0232x; 1.0232x over previous
import functools

import jax
import jax.numpy as jnp
from jax import lax
from jax.experimental import pallas as pl
from jax.experimental.pallas import tpu as pltpu
from jax.experimental.pallas import tpu_sc as plsc

NODE_DIM = 128
EDGE_DIM = 16
HIDDEN_DIM = 128
N = 10000
E = 320000

NC = 2
NS = 16
NW = NC * NS
PER_W = E // NW
CHUNK = 128
K = 79
TAIL = PER_W - (K - 1) * CHUNK
PADW = K * CHUNK
PADK = 80
NPAIR = 40
N_PAD = 10240
ROWS_PER_SUB = N_PAD // NS

_sc_mesh = plsc.VectorSubcoreMesh(core_axis_name="c", subcore_axis_name="s")
_sc_params = pltpu.CompilerParams(use_tc_tiling_on_sc=False)


def _proj_body(nf_ref, w_ref, psrc_ref, pdst_ref):
    nf = nf_ref[...]
    psrc_ref[...] = jnp.dot(nf, w_ref[0:NODE_DIM, :],
                            preferred_element_type=jnp.float32)
    pdst_ref[...] = jnp.dot(nf, w_ref[NODE_DIM:2 * NODE_DIM, :],
                            preferred_element_type=jnp.float32)


def _project_nodes(nf, eW1ab):
    return pl.pallas_call(
        _proj_body,
        out_shape=(
            jax.ShapeDtypeStruct((N, HIDDEN_DIM), jnp.float32),
            jax.ShapeDtypeStruct((N, HIDDEN_DIM), jnp.float32),
        ),
    )(nf, eW1ab)


@functools.partial(
    pl.kernel,
    out_type=(
        jax.ShapeDtypeStruct((E, HIDDEN_DIM), jnp.float32),
        jax.ShapeDtypeStruct((E, HIDDEN_DIM), jnp.float32),
    ),
    mesh=_sc_mesh,
    scratch_types=[
        pltpu.VMEM((PADK, CHUNK), jnp.int32),
        pltpu.VMEM((PADK, CHUNK), jnp.int32),
        pltpu.VMEM((CHUNK, HIDDEN_DIM), jnp.float32),
        pltpu.VMEM((CHUNK, HIDDEN_DIM), jnp.float32),
        pltpu.VMEM((CHUNK, HIDDEN_DIM), jnp.float32),
        pltpu.VMEM((CHUNK, HIDDEN_DIM), jnp.float32),
        pltpu.SemaphoreType.DMA,
        pltpu.SemaphoreType.DMA,
        pltpu.SemaphoreType.DMA,
        pltpu.SemaphoreType.DMA,
    ],
)
def _sc_gather(psrc_hbm, pdst_hbm, sidx_hbm, didx_hbm, gsrc_hbm, gdst_hbm,
               sidx_v, didx_v, a0, a1, b0, b1, sa0, sa1, sb0, sb1):
    wid = lax.axis_index("s") * NC + lax.axis_index("c")
    base = wid * PER_W
    pltpu.sync_copy(sidx_hbm.at[pl.ds(wid * PADK, PADK)], sidx_v)
    pltpu.sync_copy(didx_hbm.at[pl.ds(wid * PADK, PADK)], didx_v)

    def start(j, abuf, bbuf, sa, sb):
        pltpu.async_copy(psrc_hbm.at[sidx_v.at[j]], abuf, sa)
        pltpu.async_copy(pdst_hbm.at[didx_v.at[j]], bbuf, sb)

    def drain(j, abuf, bbuf, sa, sb):
        pltpu.make_async_copy(psrc_hbm.at[sidx_v.at[j]], abuf, sa).wait()

        @pl.when(j == K - 1)
        def _():
            pltpu.sync_copy(abuf.at[pl.ds(0, TAIL)],
                            gsrc_hbm.at[pl.ds(base + j * CHUNK, TAIL)])

        @pl.when(j != K - 1)
        def _():
            pltpu.sync_copy(abuf, gsrc_hbm.at[pl.ds(base + j * CHUNK, CHUNK)])

        pltpu.make_async_copy(pdst_hbm.at[didx_v.at[j]], bbuf, sb).wait()

        @pl.when(j == K - 1)
        def _():
            pltpu.sync_copy(bbuf.at[pl.ds(0, TAIL)],
                            gdst_hbm.at[pl.ds(base + j * CHUNK, TAIL)])

        @pl.when(j != K - 1)
        def _():
            pltpu.sync_copy(bbuf, gdst_hbm.at[pl.ds(base + j * CHUNK, CHUNK)])

    start(0, a0, b0, sa0, sb0)

    def pair(t, _):
        j0 = 2 * t

        @pl.when(t < NPAIR - 1)
        def _():
            start(j0 + 1, a1, b1, sa1, sb1)

        drain(j0, a0, b0, sa0, sb0)

        @pl.when(t < NPAIR - 1)
        def _():
            start(j0 + 2, a0, b0, sa0, sb0)
            drain(j0 + 1, a1, b1, sa1, sb1)

        return 0

    lax.fori_loop(0, NPAIR, pair, 0)


def _edge_body(gsrc_ref, gdst_ref, ef_ref, w1c_ref, b1_ref, w2_ref, b2_ref,
               g_ref, bt_ref, eo_ref):
    ef = ef_ref[...]
    pre = jnp.dot(ef, w1c_ref[...], preferred_element_type=jnp.float32)
    z = gsrc_ref[...] + gdst_ref[...] + pre + b1_ref[...]
    h1 = z * jax.nn.sigmoid(z)
    h = jnp.dot(h1, w2_ref[...], preferred_element_type=jnp.float32) + b2_ref[...]
    x = ef + h
    ones_m = jnp.full((EDGE_DIM, EDGE_DIM), 1.0 / EDGE_DIM, jnp.float32)
    m = jnp.dot(x, ones_m, preferred_element_type=jnp.float32)
    x2m = jnp.dot(x * x, ones_m, preferred_element_type=jnp.float32)
    v = x2m - m * m
    eo_ref[...] = (x - m) * lax.rsqrt(v + 1e-5) * g_ref[...] + bt_ref[...]


_EB = 8000


def _edge_mlp(gsrc, gdst, ef, eW1c, eb1, eW2, eb2, eg, ebt):
    nb = E // _EB
    return pl.pallas_call(
        _edge_body,
        grid=(nb,),
        in_specs=[
            pl.BlockSpec((_EB, HIDDEN_DIM), lambda i: (i, 0)),
            pl.BlockSpec((_EB, HIDDEN_DIM), lambda i: (i, 0)),
            pl.BlockSpec((_EB, EDGE_DIM), lambda i: (i, 0)),
            pl.BlockSpec((EDGE_DIM, HIDDEN_DIM), lambda i: (0, 0)),
            pl.BlockSpec((1, HIDDEN_DIM), lambda i: (0, 0)),
            pl.BlockSpec((HIDDEN_DIM, EDGE_DIM), lambda i: (0, 0)),
            pl.BlockSpec((1, EDGE_DIM), lambda i: (0, 0)),
            pl.BlockSpec((1, EDGE_DIM), lambda i: (0, 0)),
            pl.BlockSpec((1, EDGE_DIM), lambda i: (0, 0)),
        ],
        out_specs=pl.BlockSpec((_EB, EDGE_DIM), lambda i: (i, 0)),
        out_shape=jax.ShapeDtypeStruct((E, EDGE_DIM), jnp.float32),
    )(gsrc, gdst, ef, eW1c, eb1, eW2, eb2, eg, ebt)


@functools.partial(
    pl.kernel,
    out_type=jax.ShapeDtypeStruct((NC, N_PAD, EDGE_DIM), jnp.float32),
    mesh=_sc_mesh,
    scratch_types=[
        pltpu.VMEM((PADK, CHUNK), jnp.int32),
        pltpu.VMEM((CHUNK, EDGE_DIM), jnp.float32),
        pltpu.VMEM((CHUNK, EDGE_DIM), jnp.float32),
        pltpu.VMEM_SHARED((N_PAD, EDGE_DIM), jnp.float32),
        pltpu.SemaphoreType.DMA,
        pltpu.SemaphoreType.DMA,
    ],
    compiler_params=_sc_params,
)
def _sc_scatter(eo_hbm, didx_hbm, zeros_hbm, agg_hbm, didx_v, r0, r1, table,
                s0, s1):
    c = lax.axis_index("c")
    s = lax.axis_index("s")
    wid = s * NC + c
    base = wid * PER_W
    pltpu.sync_copy(zeros_hbm.at[pl.ds(s * ROWS_PER_SUB, ROWS_PER_SUB)],
                    table.at[pl.ds(s * ROWS_PER_SUB, ROWS_PER_SUB)])
    pltpu.sync_copy(didx_hbm.at[pl.ds(wid * PADK, PADK)], didx_v)
    plsc.subcore_barrier()

    def start(j, rbuf, sem):
        @pl.when(j == K - 1)
        def _():
            pltpu.async_copy(eo_hbm.at[pl.ds(base + j * CHUNK, TAIL)],
                             rbuf.at[pl.ds(0, TAIL)], sem)

        @pl.when(j != K - 1)
        def _():
            pltpu.async_copy(eo_hbm.at[pl.ds(base + j * CHUNK, CHUNK)],
                             rbuf, sem)

    def drain(j, rbuf, sem):
        @pl.when(j == K - 1)
        def _():
            pltpu.make_async_copy(eo_hbm.at[pl.ds(base + j * CHUNK, TAIL)],
                                  rbuf.at[pl.ds(0, TAIL)], sem).wait()

        @pl.when(j != K - 1)
        def _():
            pltpu.make_async_copy(eo_hbm.at[pl.ds(base + j * CHUNK, CHUNK)],
                                  rbuf, sem).wait()

        pltpu.sync_copy(rbuf, table.at[didx_v.at[j]], add=True)

    start(0, r0, s0)

    def pair(t, _):
        j0 = 2 * t

        @pl.when(t < NPAIR - 1)
        def _():
            start(j0 + 1, r1, s1)

        drain(j0, r0, s0)

        @pl.when(t < NPAIR - 1)
        def _():
            start(j0 + 2, r0, s0)
            drain(j0 + 1, r1, s1)

        return 0

    lax.fori_loop(0, NPAIR, pair, 0)
    plsc.subcore_barrier()
    pltpu.sync_copy(table.at[pl.ds(s * ROWS_PER_SUB, ROWS_PER_SUB)],
                    agg_hbm.at[c, pl.ds(s * ROWS_PER_SUB, ROWS_PER_SUB)])


def _node_body(nf_ref, a0_ref, a1_ref, w1a_ref, w1b_ref, b1_ref, w2_ref,
               b2_ref, g_ref, bt_ref, out_ref):
    nf = nf_ref[...]
    agg = a0_ref[...] + a1_ref[...]
    z = (jnp.dot(nf, w1a_ref[...], preferred_element_type=jnp.float32)
         + jnp.dot(agg, w1b_ref[...], preferred_element_type=jnp.float32)
         + b1_ref[...])
    h1 = z * jax.nn.sigmoid(z)
    h2 = jnp.dot(h1, w2_ref[...], preferred_element_type=jnp.float32) + b2_ref[...]
    x = nf + h2
    ones_m = jnp.full((NODE_DIM, NODE_DIM), 1.0 / NODE_DIM, jnp.float32)
    m = jnp.dot(x, ones_m, preferred_element_type=jnp.float32)
    x2m = jnp.dot(x * x, ones_m, preferred_element_type=jnp.float32)
    v = x2m - m * m
    out_ref[...] = (x - m) * lax.rsqrt(v + 1e-5) * g_ref[...] + bt_ref[...]


_NB = 2000


def _node_mlp(nf, a0, a1, nW1a, nW1b, nb1, nW2, nb2, ng, nbt):
    nb = N // _NB
    return pl.pallas_call(
        _node_body,
        grid=(nb,),
        in_specs=[
            pl.BlockSpec((_NB, NODE_DIM), lambda i: (i, 0)),
            pl.BlockSpec((_NB, EDGE_DIM), lambda i: (i, 0)),
            pl.BlockSpec((_NB, EDGE_DIM), lambda i: (i, 0)),
            pl.BlockSpec((NODE_DIM, HIDDEN_DIM), lambda i: (0, 0)),
            pl.BlockSpec((EDGE_DIM, HIDDEN_DIM), lambda i: (0, 0)),
            pl.BlockSpec((1, HIDDEN_DIM), lambda i: (0, 0)),
            pl.BlockSpec((HIDDEN_DIM, NODE_DIM), lambda i: (0, 0)),
            pl.BlockSpec((1, NODE_DIM), lambda i: (0, 0)),
            pl.BlockSpec((1, NODE_DIM), lambda i: (0, 0)),
            pl.BlockSpec((1, NODE_DIM), lambda i: (0, 0)),
        ],
        out_specs=pl.BlockSpec((_NB, NODE_DIM), lambda i: (i, 0)),
        out_shape=jax.ShapeDtypeStruct((N, NODE_DIM), jnp.float32),
    )(nf, a0, a1, nW1a, nW1b, nb1, nW2, nb2, ng, nbt)


def _worker_idx(idx, pad_value):
    arr = idx.reshape(NW, PER_W)
    arr = jnp.pad(arr, ((0, 0), (0, PADK * CHUNK - PER_W)),
                  constant_values=pad_value)
    return arr.reshape(NW * PADK, CHUNK)


@jax.jit
def kernel(node_feat, edge_feat, edge_index, eW1, eb1, eW2, eb2, eg, ebt,
           nW1, nb1, nW2, nb2, ng, nbt):
    nf = node_feat[0]
    ef = edge_feat[0]
    src = edge_index[0].astype(jnp.int32)
    dst = edge_index[1].astype(jnp.int32)

    sidx = _worker_idx(src, 0)
    didx_g = _worker_idx(dst, 0)
    didx_s = _worker_idx(dst, N)

    psrc, pdst = _project_nodes(nf, eW1[0:2 * NODE_DIM])
    gsrc, gdst = _sc_gather(psrc, pdst, sidx, didx_g)
    eo = _edge_mlp(gsrc, gdst, ef, eW1[2 * NODE_DIM:],
                   eb1.reshape(1, -1), eW2, eb2.reshape(1, -1),
                   eg.reshape(1, -1), ebt.reshape(1, -1))
    zeros = jnp.zeros((N_PAD, EDGE_DIM), jnp.float32)
    agg2 = _sc_scatter(eo, didx_s, zeros)
    node_out = _node_mlp(nf, agg2[0, :N], agg2[1, :N],
                         nW1[0:NODE_DIM], nW1[NODE_DIM:],
                         nb1.reshape(1, -1), nW2, nb2.reshape(1, -1),
                         ng.reshape(1, -1), nbt.reshape(1, -1))
    return (node_out[None], eo[None])
